# Initial kernel scaffold; baseline (speedup 1.0000x reference)
import jax
import jax.numpy as jnp
from jax.experimental import pallas as pl


def kernel(x, edge_index, edge_index2, l0_Wq, l0_bq, l0_Wk, l0_bk, l0_Wv, l0_bv, l1_Wq, l1_bq, l1_Wk, l1_bk, l1_Wv, l1_bv):
    raise NotImplementedError("write your pallas kernel here")


# speedup vs baseline: 19.7510x; 19.7510x over previous
import functools
import math

import jax
import jax.numpy as jnp
from jax import lax
from jax.experimental import pallas as pl
from jax.experimental.pallas import tpu as pltpu
from jax.experimental.pallas import tpu_sc as plsc

NN = 10000
EE = 320000
EPAD = 321536
HEADS = 4
NC, NS, LANES = 2, 16, 16
NP = 10240
RPT = NP // NS
RB = 1000


def _c16(v):
    return jnp.full((LANES,), v, jnp.int32)


@functools.cache
def _make_edge_kernel(qw, kvw, accw, d_k, chunk):
    dc = HEADS * d_k
    voff = kvw // 2
    inv = 1.0 / math.sqrt(d_k)
    cpt = EPAD // chunk // NS
    mesh = plsc.VectorSubcoreMesh(core_axis_name="c", subcore_axis_name="s",
                                  num_cores=NC, num_subcores=NS)

    @functools.partial(
        pl.kernel,
        out_type=jax.ShapeDtypeStruct((NC * NP, accw), jnp.float32),
        mesh=mesh,
        compiler_params=pltpu.CompilerParams(needs_layout_passes=False,
                                             use_tc_tiling_on_sc=False),
        scratch_types=[
            pltpu.VMEM((chunk, qw), jnp.float32),
            pltpu.VMEM((chunk, kvw), jnp.float32),
            pltpu.VMEM((chunk, accw), jnp.float32),
            pltpu.VMEM((chunk,), jnp.int32),
            pltpu.VMEM((chunk,), jnp.int32),
            pltpu.VMEM((chunk,), jnp.int32),
            pltpu.VMEM_SHARED((NP, accw), jnp.float32),
            pltpu.SemaphoreType.DMA,
            pltpu.SemaphoreType.DMA,
        ],
    )
    def edge_kernel(q_hbm, kv_hbm, gsrc_hbm, gdst_hbm, rsrc_hbm, out_hbm,
                    qv, kvv, msgv, gsv, gdv, rsv, acc, sem1, sem2):
        c = lax.axis_index("c")
        s = lax.axis_index("s")
        zero16 = jnp.zeros((LANES,), jnp.float32)

        for r in range(chunk):
            for j in range(accw // LANES):
                msgv[r, pl.ds(j * LANES, LANES)] = zero16

        base = s * RPT
        for kb in range(RPT // chunk):
            pltpu.sync_copy(msgv, acc.at[pl.ds(base + kb * chunk, chunk)])
        plsc.subcore_barrier()

        def chunk_body(i, carry):
            off = c * EPAD + (s * cpt + i) * chunk
            pltpu.sync_copy(gsrc_hbm.at[pl.ds(off, chunk)], gsv)
            pltpu.sync_copy(gdst_hbm.at[pl.ds(off, chunk)], gdv)
            pltpu.sync_copy(rsrc_hbm.at[pl.ds(off, chunk)], rsv)
            cp1 = pltpu.async_copy(q_hbm.at[gsv], qv, sem1)
            cp2 = pltpu.async_copy(kv_hbm.at[gdv], kvv, sem2)
            cp1.wait()
            cp2.wait()

            def group(g, gcarry):
                rows = lax.iota(jnp.int32, LANES) + g * LANES
                for h in range(HEADS):
                    p = jnp.zeros((LANES,), jnp.float32)
                    for j in range(d_k):
                        col = h * d_k + j
                        qc = plsc.load_gather(qv, [rows, _c16(col)])
                        kc = plsc.load_gather(kvv, [rows, _c16(col)])
                        p = p + qc * kc
                    w = jnp.exp(p * inv)
                    plsc.store_scatter(msgv, [rows, _c16(dc + h)], w)
                    for j in range(d_k):
                        col = h * d_k + j
                        vc = plsc.load_gather(kvv, [rows, _c16(voff + col)])
                        plsc.store_scatter(msgv, [rows, _c16(col)], w * vc)
                return gcarry
            lax.fori_loop(0, chunk // LANES, group, 0)
            pltpu.sync_copy(msgv, acc.at[rsv], add=True)
            return carry
        lax.fori_loop(0, cpt, chunk_body, 0)
        plsc.subcore_barrier()

        for kb in range(RPT // chunk):
            r0 = base + kb * chunk
            pltpu.sync_copy(acc.at[pl.ds(r0, chunk)], msgv)
            pltpu.sync_copy(msgv, out_hbm.at[pl.ds(c * NP + r0, chunk)])

    return edge_kernel


def _edge_l0(*args):
    return _make_edge_kernel(qw=128, kvw=256, accw=144, d_k=32, chunk=64)(*args)


def _edge_l1(*args):
    return _make_edge_kernel(qw=48, kvw=96, accw=48, d_k=10, chunk=128)(*args)


def _l0_proj_body(x_ref, w_ref, b_ref, q_ref, kv_ref):
    y = jnp.dot(x_ref[...], w_ref[0], preferred_element_type=jnp.float32)
    y = y + b_ref[0]
    q_ref[...] = y[:, :128]
    kv_ref[...] = y[:, 128:]


def _l0_proj(x, w0, b0):
    gn = NN // RB
    return pl.pallas_call(
        _l0_proj_body,
        grid=(NC, gn),
        in_specs=[
            pl.BlockSpec((RB, 128), lambda i, j: (j, 0)),
            pl.BlockSpec((1, 128, 384), lambda i, j: (i, 0, 0)),
            pl.BlockSpec((1, 1, 384), lambda i, j: (i, 0, 0)),
        ],
        out_specs=[
            pl.BlockSpec((RB, 128), lambda i, j: (i * gn + j, 0)),
            pl.BlockSpec((RB, 256), lambda i, j: (i * gn + j, 0)),
        ],
        out_shape=[
            jax.ShapeDtypeStruct((NC * NN, 128), jnp.float32),
            jax.ShapeDtypeStruct((NC * NN, 256), jnp.float32),
        ],
    )(x, w0, b0)


def _combine(a0, a1, e4, dcw):
    d_a = jnp.dot(a0[:, dcw:dcw + HEADS], e4, preferred_element_type=jnp.float32)
    d_b = jnp.dot(a1[:, dcw:dcw + HEADS], e4, preferred_element_type=jnp.float32)
    return 0.5 * (a0[:, :dcw] / (d_a + 1e-16) + 0.5 * (a1[:, :dcw] / (d_b + 1e-16)))


def _l1_proj_body(acc_ref, w_ref, b_ref, e4_ref, q_ref, kv_ref):
    h = _combine(acc_ref[0], acc_ref[1], e4_ref[...], 128)
    h = jnp.where(h > 0, h, jnp.exp(jnp.minimum(h, 0.0)) - 1.0)
    y = jnp.dot(h, w_ref[0], preferred_element_type=jnp.float32)
    y = y + b_ref[0]
    q_ref[...] = y[:, :48]
    kv_ref[...] = y[:, 48:]


def _l1_proj(acc0, w1, b1, e4):
    gn = NN // RB
    return pl.pallas_call(
        _l1_proj_body,
        grid=(NC, gn),
        in_specs=[
            pl.BlockSpec((NC, RB, 144), lambda i, j: (0, j, 0)),
            pl.BlockSpec((1, 128, 144), lambda i, j: (i, 0, 0)),
            pl.BlockSpec((1, 1, 144), lambda i, j: (i, 0, 0)),
            pl.BlockSpec((HEADS, 128), lambda i, j: (0, 0)),
        ],
        out_specs=[
            pl.BlockSpec((RB, 48), lambda i, j: (i * gn + j, 0)),
            pl.BlockSpec((RB, 96), lambda i, j: (i * gn + j, 0)),
        ],
        out_shape=[
            jax.ShapeDtypeStruct((NC * NN, 48), jnp.float32),
            jax.ShapeDtypeStruct((NC * NN, 96), jnp.float32),
        ],
    )(acc0, w1, b1, e4)


def _final_body(acc_ref, e4_ref, o_ref):
    o = _combine(acc_ref[0], acc_ref[1], e4_ref[...], 40)
    m = jnp.max(o, axis=-1, keepdims=True)
    z = o - m
    lse = jnp.log(jnp.sum(jnp.exp(z), axis=-1, keepdims=True))
    o_ref[...] = z - lse


def _final(acc1, e4):
    gn = NN // RB
    return pl.pallas_call(
        _final_body,
        grid=(gn,),
        in_specs=[
            pl.BlockSpec((NC, RB, 48), lambda j: (0, j, 0)),
            pl.BlockSpec((HEADS, 40), lambda j: (0, 0)),
        ],
        out_specs=pl.BlockSpec((RB, 40), lambda j: (j, 0)),
        out_shape=jax.ShapeDtypeStruct((NN, 40), jnp.float32),
    )(acc1, e4)


def kernel(x, edge_index, edge_index2, l0_Wq, l0_bq, l0_Wk, l0_bk, l0_Wv,
           l0_bv, l1_Wq, l1_bq, l1_Wk, l1_bk, l1_Wv, l1_bv):
    ei = edge_index.astype(jnp.int32)
    ei2 = edge_index2.astype(jnp.int32)
    padg = jnp.zeros((EPAD - EE,), jnp.int32)
    padr = jnp.full((EPAD - EE,), NP - 1, jnp.int32)
    gsrc = jnp.concatenate([ei[0], padg, ei2[0] + NN, padg + NN])
    gdst = jnp.concatenate([ei[1], padg, ei2[1] + NN, padg + NN])
    rsrc = jnp.concatenate([ei[0], padr, ei2[0], padr])

    w0 = jnp.concatenate([jnp.transpose(l0_Wq, (0, 2, 1)),
                          jnp.transpose(l0_Wk, (0, 2, 1)),
                          jnp.transpose(l0_Wv, (0, 2, 1))], axis=2)
    b0 = jnp.concatenate([l0_bq, l0_bk, l0_bv], axis=1)[:, None, :]
    q0, kv0 = _l0_proj(x, w0, b0)

    acc0 = _edge_l0(q0, kv0, gsrc, gdst, rsrc).reshape(NC, NP, 144)[:, :NN]

    pad8 = jnp.zeros((NC, 8, 128), jnp.float32)
    padb = jnp.zeros((NC, 8), jnp.float32)
    w1 = jnp.concatenate([l1_Wq, pad8, l1_Wk, pad8, l1_Wv, pad8], axis=1)
    w1 = jnp.transpose(w1, (0, 2, 1))
    b1 = jnp.concatenate([l1_bq, padb, l1_bk, padb, l1_bv, padb], axis=1)[:, None, :]
    e4a = jnp.repeat(jnp.eye(HEADS, dtype=jnp.float32), 32, axis=1)
    q1, kv1 = _l1_proj(acc0, w1, b1, e4a)

    acc1 = _edge_l1(q1, kv1, gsrc, gdst, rsrc).reshape(NC, NP, 48)[:, :NN]
    e4b = jnp.repeat(jnp.eye(HEADS, dtype=jnp.float32), 10, axis=1)
    return _final(acc1, e4b)


# speedup vs baseline: 31.9175x; 1.6160x over previous
import functools
import math

import jax
import jax.numpy as jnp
from jax import lax
from jax.experimental import pallas as pl
from jax.experimental.pallas import tpu as pltpu
from jax.experimental.pallas import tpu_sc as plsc

NN = 10000
EE = 320000
EPAD = 321536
HEADS = 4
NC, NS, LANES = 2, 16, 16
NP = 10240
RPT = NP // NS
RB = 1000


def _c16(v):
    return jnp.full((LANES,), v, jnp.int32)


@functools.cache
def _make_edge_kernel(qw, kvw, voff, accw, d_k, chunk):
    dc = HEADS * d_k
    inv = 1.0 / math.sqrt(d_k)
    cpt = EPAD // chunk // NS
    mesh = plsc.VectorSubcoreMesh(core_axis_name="c", subcore_axis_name="s",
                                  num_cores=NC, num_subcores=NS)

    @functools.partial(
        pl.kernel,
        out_type=jax.ShapeDtypeStruct((NC * NP, accw), jnp.float32),
        mesh=mesh,
        compiler_params=pltpu.CompilerParams(needs_layout_passes=False,
                                             use_tc_tiling_on_sc=False),
        scratch_types=[
            pltpu.VMEM((chunk, qw), jnp.float32),
            pltpu.VMEM((chunk, kvw), jnp.float32),
            pltpu.VMEM((chunk, accw), jnp.float32),
            pltpu.VMEM((chunk,), jnp.int32),
            pltpu.VMEM((chunk,), jnp.int32),
            pltpu.VMEM((chunk,), jnp.int32),
            pltpu.VMEM_SHARED((NP, accw), jnp.float32),
            pltpu.SemaphoreType.DMA,
            pltpu.SemaphoreType.DMA,
        ],
    )
    def edge_kernel(q_hbm, kv_hbm, gsrc_hbm, gdst_hbm, rsrc_hbm, out_hbm,
                    qv, kvv, msgv, gsv, gdv, rsv, acc, sem1, sem2):
        c = lax.axis_index("c")
        s = lax.axis_index("s")
        zero16 = jnp.zeros((LANES,), jnp.float32)

        for r in range(chunk):
            for j in range(accw // LANES):
                msgv[r, pl.ds(j * LANES, LANES)] = zero16

        base = s * RPT
        for kb in range(RPT // chunk):
            pltpu.sync_copy(msgv, acc.at[pl.ds(base + kb * chunk, chunk)])
        plsc.subcore_barrier()

        def chunk_body(i, carry):
            off = c * EPAD + (s * cpt + i) * chunk
            ci1 = pltpu.async_copy(gsrc_hbm.at[pl.ds(off, chunk)], gsv, sem1)
            ci2 = pltpu.async_copy(gdst_hbm.at[pl.ds(off, chunk)], gdv, sem1)
            ci3 = pltpu.async_copy(rsrc_hbm.at[pl.ds(off, chunk)], rsv, sem1)
            ci1.wait()
            ci2.wait()
            ci3.wait()
            cp1 = pltpu.async_copy(q_hbm.at[gsv], qv, sem1)
            cp2 = pltpu.async_copy(kv_hbm.at[gdv], kvv, sem2)
            cp1.wait()
            cp2.wait()

            def group(g, gcarry):
                rows = lax.iota(jnp.int32, LANES) + g * LANES
                for h in range(HEADS):
                    p = jnp.zeros((LANES,), jnp.float32)
                    for j in range(d_k):
                        col = h * d_k + j
                        qc = plsc.load_gather(qv, [rows, _c16(col)])
                        kc = plsc.load_gather(kvv, [rows, _c16(col)])
                        p = p + qc * kc
                    w = jnp.exp(p * inv)
                    plsc.store_scatter(msgv, [rows, _c16(dc + h)], w)
                    for j in range(d_k):
                        col = h * d_k + j
                        vc = plsc.load_gather(kvv, [rows, _c16(voff + col)])
                        plsc.store_scatter(msgv, [rows, _c16(col)], w * vc)
                return gcarry
            lax.fori_loop(0, chunk // LANES, group, 0)
            pltpu.sync_copy(msgv, acc.at[rsv], add=True)
            return carry
        lax.fori_loop(0, cpt, chunk_body, 0)
        plsc.subcore_barrier()

        for kb in range(RPT // chunk):
            r0 = base + kb * chunk
            pltpu.sync_copy(acc.at[pl.ds(r0, chunk)], msgv)
            pltpu.sync_copy(msgv, out_hbm.at[pl.ds(c * NP + r0, chunk)])

    return edge_kernel


def _edge_l0(*args):
    return _make_edge_kernel(qw=144, kvw=272, voff=136, accw=144, d_k=32,
                             chunk=64)(*args)


def _edge_l1(*args):
    return _make_edge_kernel(qw=48, kvw=112, voff=56, accw=48, d_k=10,
                             chunk=128)(*args)


def _l0_proj_body(x_ref, w_ref, b_ref, q_ref, kv_ref):
    y = jnp.dot(x_ref[...], w_ref[0], preferred_element_type=jnp.float32)
    y = y + b_ref[0]
    q_ref[...] = y[:, :144]
    kv_ref[...] = y[:, 144:]


def _l0_proj(x, w0, b0):
    gn = NN // RB
    return pl.pallas_call(
        _l0_proj_body,
        grid=(NC, gn),
        in_specs=[
            pl.BlockSpec((RB, 128), lambda i, j: (j, 0)),
            pl.BlockSpec((1, 128, 416), lambda i, j: (i, 0, 0)),
            pl.BlockSpec((1, 1, 416), lambda i, j: (i, 0, 0)),
        ],
        out_specs=[
            pl.BlockSpec((RB, 144), lambda i, j: (i * gn + j, 0)),
            pl.BlockSpec((RB, 272), lambda i, j: (i * gn + j, 0)),
        ],
        out_shape=[
            jax.ShapeDtypeStruct((NC * NN, 144), jnp.float32),
            jax.ShapeDtypeStruct((NC * NN, 272), jnp.float32),
        ],
    )(x, w0, b0)


def _combine(a0, a1, e4, dcw):
    d_a = jnp.dot(a0[:, dcw:dcw + HEADS], e4, preferred_element_type=jnp.float32)
    d_b = jnp.dot(a1[:, dcw:dcw + HEADS], e4, preferred_element_type=jnp.float32)
    return 0.5 * (a0[:, :dcw] / (d_a + 1e-16) + 0.5 * (a1[:, :dcw] / (d_b + 1e-16)))


def _l1_proj_body(acc_ref, w_ref, b_ref, e4_ref, q_ref, kv_ref):
    h = _combine(acc_ref[0], acc_ref[1], e4_ref[...], 128)
    h = jnp.where(h > 0, h, jnp.exp(jnp.minimum(h, 0.0)) - 1.0)
    y = jnp.dot(h, w_ref[0], preferred_element_type=jnp.float32)
    y = y + b_ref[0]
    q_ref[...] = y[:, :48]
    kv_ref[...] = y[:, 48:160]


def _l1_proj(acc0, w1, b1, e4):
    gn = NN // RB
    return pl.pallas_call(
        _l1_proj_body,
        grid=(NC, gn),
        in_specs=[
            pl.BlockSpec((NC, RB, 144), lambda i, j: (0, j, 0)),
            pl.BlockSpec((1, 128, 160), lambda i, j: (i, 0, 0)),
            pl.BlockSpec((1, 1, 160), lambda i, j: (i, 0, 0)),
            pl.BlockSpec((HEADS, 128), lambda i, j: (0, 0)),
        ],
        out_specs=[
            pl.BlockSpec((RB, 48), lambda i, j: (i * gn + j, 0)),
            pl.BlockSpec((RB, 112), lambda i, j: (i * gn + j, 0)),
        ],
        out_shape=[
            jax.ShapeDtypeStruct((NC * NN, 48), jnp.float32),
            jax.ShapeDtypeStruct((NC * NN, 112), jnp.float32),
        ],
    )(acc0, w1, b1, e4)


def _final_body(acc_ref, e4_ref, o_ref):
    o = _combine(acc_ref[0], acc_ref[1], e4_ref[...], 40)
    m = jnp.max(o, axis=-1, keepdims=True)
    z = o - m
    lse = jnp.log(jnp.sum(jnp.exp(z), axis=-1, keepdims=True))
    o_ref[...] = z - lse


def _final(acc1, e4):
    gn = NN // RB
    return pl.pallas_call(
        _final_body,
        grid=(gn,),
        in_specs=[
            pl.BlockSpec((NC, RB, 48), lambda j: (0, j, 0)),
            pl.BlockSpec((HEADS, 40), lambda j: (0, 0)),
        ],
        out_specs=pl.BlockSpec((RB, 40), lambda j: (j, 0)),
        out_shape=jax.ShapeDtypeStruct((NN, 40), jnp.float32),
    )(acc1, e4)


def kernel(x, edge_index, edge_index2, l0_Wq, l0_bq, l0_Wk, l0_bk, l0_Wv,
           l0_bv, l1_Wq, l1_bq, l1_Wk, l1_bk, l1_Wv, l1_bv):
    ei = edge_index.astype(jnp.int32)
    ei2 = edge_index2.astype(jnp.int32)
    padg = jnp.zeros((EPAD - EE,), jnp.int32)
    padr = jnp.full((EPAD - EE,), NP - 1, jnp.int32)
    gsrc = jnp.concatenate([ei[0], padg, ei2[0] + NN, padg + NN])
    gdst = jnp.concatenate([ei[1], padg, ei2[1] + NN, padg + NN])
    rsrc = jnp.concatenate([ei[0], padr, ei2[0], padr])

    zw8 = jnp.zeros((NC, 128, 8), jnp.float32)
    zb8 = jnp.zeros((NC, 8), jnp.float32)
    w0 = jnp.concatenate([jnp.transpose(l0_Wq, (0, 2, 1)), zw8, zw8,
                          jnp.transpose(l0_Wk, (0, 2, 1)), zw8,
                          jnp.transpose(l0_Wv, (0, 2, 1)), zw8], axis=2)
    b0 = jnp.concatenate([l0_bq, zb8, zb8, l0_bk, zb8, l0_bv, zb8],
                         axis=1)[:, None, :]
    q0, kv0 = _l0_proj(x, w0, b0)

    acc0 = _edge_l0(q0, kv0, gsrc, gdst, rsrc).reshape(NC, NP, 144)[:, :NN]

    w1 = jnp.concatenate([jnp.transpose(l1_Wq, (0, 2, 1)), zw8,
                          jnp.transpose(l1_Wk, (0, 2, 1)), zw8, zw8,
                          jnp.transpose(l1_Wv, (0, 2, 1)), zw8, zw8], axis=2)
    b1 = jnp.concatenate([l1_bq, zb8, l1_bk, zb8, zb8, l1_bv, zb8, zb8],
                         axis=1)[:, None, :]
    e4a = jnp.repeat(jnp.eye(HEADS, dtype=jnp.float32), 32, axis=1)
    q1, kv1 = _l1_proj(acc0, w1, b1, e4a)

    acc1 = _edge_l1(q1, kv1, gsrc, gdst, rsrc).reshape(NC, NP, 48)[:, :NN]
    e4b = jnp.repeat(jnp.eye(HEADS, dtype=jnp.float32), 10, axis=1)
    return _final(acc1, e4b)
